# baseline (speedup 1.0000x reference)
import functools

import jax
import jax.numpy as jnp
from jax import lax
from jax.experimental import pallas as pl
from jax.experimental.pallas import tpu as pltpu
from jax.experimental.pallas import tpu_sc as plsc

_NC, _NS = 2, 16
_NW = _NC * _NS


def _routing_body(xq_ref, ek_ref, tc_ref, idx_ref):
    xq = xq_ref[...]
    ek = ek_ref[...]
    tc = tc_ref[...]
    nk = ek / jnp.clip(
        jnp.sqrt(jnp.sum(ek * ek, axis=1, keepdims=True)), 1e-12, None)
    qn = xq / jnp.clip(
        jnp.sqrt(jnp.sum(xq * xq, axis=1, keepdims=True)), 1e-12, None)
    cos = lax.dot_general(
        qn, nk, (((1,), (1,)), ((), ())),
        preferred_element_type=jnp.float32)
    scaled = (1.0 - cos) * (tc + 1.0)
    B, E = scaled.shape
    col = lax.broadcasted_iota(jnp.int32, (B, E), 1)
    m0 = jnp.min(scaled, axis=1, keepdims=True)
    i0 = jnp.min(jnp.where(scaled == m0, col, E), axis=1, keepdims=True)
    masked = jnp.where(col == i0, jnp.inf, scaled)
    m1 = jnp.min(masked, axis=1, keepdims=True)
    i1 = jnp.min(jnp.where(masked == m1, col, E), axis=1, keepdims=True)
    c = lax.broadcasted_iota(jnp.int32, (B, 24), 1)
    half = c // 12
    j = c % 12
    erow = jnp.where(j < 4, i0 * 8 + half * 4 + j,
                     i1 * 8 + half * 4 + (j - 4))
    grow = E * 8 + half * 4 + (j - 8)
    idx_ref[...] = jnp.where(j < 8, erow, grow)


def _routing_call(xq, ek, tc2d):
    B = xq.shape[0]
    return pl.pallas_call(
        _routing_body,
        out_shape=jax.ShapeDtypeStruct((B, 24), jnp.int32),
    )(xq, ek, tc2d)


@functools.lru_cache(maxsize=None)
def _gather_kernel(B, D):
    ppw = B // _NW // 2
    nslots = 4
    mesh = plsc.VectorSubcoreMesh(core_axis_name="c", subcore_axis_name="s")

    @functools.partial(
        pl.kernel,
        out_type=(jax.ShapeDtypeStruct((B * 12, D), jnp.float32),
                  jax.ShapeDtypeStruct((B * 12, D), jnp.float32)),
        mesh=mesh,
        scratch_types=[
            pltpu.VMEM((2 * ppw, 1, 24), jnp.int32),
            pltpu.VMEM((nslots, 24, D), jnp.float32),
            [pltpu.SemaphoreType.DMA] * nslots,
            [pltpu.SemaphoreType.DMA] * nslots,
        ],
    )
    def k(tab_hbm, idx_hbm, pk_hbm, pv_hbm, idx_v, ebuf, gsems, wsems):
        wid = lax.axis_index("s") * _NC + lax.axis_index("c")
        pbase = wid * ppw
        pltpu.sync_copy(idx_hbm.at[pl.ds(2 * pbase, 2 * ppw)], idx_v)

        def issue_gather(p, s2):
            for h in range(2):
                pltpu.async_copy(tab_hbm.at[idx_v.at[2 * p + h, 0]],
                                 ebuf.at[s2 + h], gsems[s2 + h])

        def wait_gather(s):
            pltpu.make_async_copy(tab_hbm.at[idx_v.at[0, 0]],
                                  ebuf.at[s], gsems[s]).wait()

        def issue_writes(p, s2):
            row = (pbase + p) * 24
            pltpu.async_copy(ebuf.at[s2], pk_hbm.at[pl.ds(row, 24)],
                             wsems[s2])
            pltpu.async_copy(ebuf.at[s2 + 1], pv_hbm.at[pl.ds(row, 24)],
                             wsems[s2 + 1])

        def wait_writes(s):
            pltpu.make_async_copy(ebuf.at[s], pk_hbm.at[pl.ds(0, 24)],
                                  wsems[s]).wait()

        issue_gather(0, 0)
        issue_gather(1, 2)

        @pl.loop(0, ppw, step=2)
        def _(p0):
            for kk in range(2):
                wait_gather(2 * kk)
                wait_gather(2 * kk + 1)
                issue_writes(p0 + kk, 2 * kk)
            for kk in range(2):
                wait_writes(2 * kk)
                wait_writes(2 * kk + 1)
                nxt = p0 + 2 + kk

                @pl.when(nxt < ppw)
                def _():
                    issue_gather(nxt, 2 * kk)

    return k


def kernel(x_querry, l, x_block, e_k, e_p_0, g_p_0, train_count):
    B, _ = x_querry.shape
    E, PLEN, D = e_p_0.shape
    idx = _routing_call(x_querry, e_k, train_count.reshape(1, E))
    idx_pairs = (idx.reshape(B // 2, 2, 2, 12)
                 .transpose(0, 2, 1, 3).reshape(B, 1, 24))
    table = jnp.concatenate([e_p_0.reshape(E * PLEN, D), g_p_0], axis=0)
    pk, pv = _gather_kernel(B, D)(table, idx_pairs)
    return pk.reshape(B, 12, D), pv.reshape(B, 12, D), x_block


# speedup vs baseline: 1.6556x; 1.6556x over previous
import functools

import jax
import jax.numpy as jnp
from jax import lax
from jax.experimental import pallas as pl
from jax.experimental.pallas import tpu as pltpu
from jax.experimental.pallas import tpu_sc as plsc

_NC, _NS = 2, 16
_NW = _NC * _NS


def _routing_body(xq_ref, ek_ref, tc_ref, idx_ref):
    xq = xq_ref[...]
    ek = ek_ref[...]
    tc = tc_ref[...]
    nk = ek / jnp.clip(
        jnp.sqrt(jnp.sum(ek * ek, axis=1, keepdims=True)), 1e-12, None)
    qn = xq / jnp.clip(
        jnp.sqrt(jnp.sum(xq * xq, axis=1, keepdims=True)), 1e-12, None)
    cos = lax.dot_general(
        qn, nk, (((1,), (1,)), ((), ())),
        preferred_element_type=jnp.float32)
    scaled = (1.0 - cos) * (tc + 1.0)
    B, E = scaled.shape
    col = lax.broadcasted_iota(jnp.int32, (B, E), 1)
    m0 = jnp.min(scaled, axis=1, keepdims=True)
    i0 = jnp.min(jnp.where(scaled == m0, col, E), axis=1, keepdims=True)
    masked = jnp.where(col == i0, jnp.inf, scaled)
    m1 = jnp.min(masked, axis=1, keepdims=True)
    i1 = jnp.min(jnp.where(masked == m1, col, E), axis=1, keepdims=True)
    c = lax.broadcasted_iota(jnp.int32, (B, 16), 1)
    half = c // 8
    j = c % 8
    idx_ref[...] = jnp.where(j < 4, i0 * 8 + half * 4 + j,
                             i1 * 8 + half * 4 + (j - 4))


def _routing_call(xq, ek, tc2d):
    B = xq.shape[0]
    return pl.pallas_call(
        _routing_body,
        out_shape=jax.ShapeDtypeStruct((B, 16), jnp.int32),
    )(xq, ek, tc2d)


@functools.lru_cache(maxsize=None)
def _gather_kernel(B, D):
    qpw = B // _NW
    nslots = 4
    mesh = plsc.VectorSubcoreMesh(core_axis_name="c", subcore_axis_name="s")

    @functools.partial(
        pl.kernel,
        out_type=(jax.ShapeDtypeStruct((B, 12, D), jnp.float32),
                  jax.ShapeDtypeStruct((B, 12, D), jnp.float32)),
        mesh=mesh,
        scratch_types=[
            pltpu.VMEM((qpw * 16,), jnp.int32),
            pltpu.VMEM((nslots, 2, 12, D), jnp.float32),
            [pltpu.SemaphoreType.DMA] * nslots,
            [pltpu.SemaphoreType.DMA] * nslots,
        ],
    )
    def k(tab_hbm, gpad_hbm, idx_hbm, pk_hbm, pv_hbm,
          idx_v, ebuf, gsems, wsems):
        wid = lax.axis_index("s") * _NC + lax.axis_index("c")
        base = wid * qpw
        pltpu.sync_copy(idx_hbm.at[wid, 0], idx_v)
        for s in range(nslots):
            pltpu.sync_copy(gpad_hbm, ebuf.at[s])

        def issue_gather(q, s):
            for h in range(2):
                pltpu.async_copy(
                    tab_hbm.at[idx_v.at[pl.ds((2 * q + h) * 8, 8)]],
                    ebuf.at[s, h, pl.ds(0, 8)], gsems[s])

        def wait_gather(s):
            for h in range(2):
                pltpu.make_async_copy(tab_hbm.at[idx_v.at[pl.ds(0, 8)]],
                                      ebuf.at[s, h, pl.ds(0, 8)],
                                      gsems[s]).wait()

        def issue_writes(q, s):
            qa = base + q
            pltpu.async_copy(ebuf.at[s, 0], pk_hbm.at[qa], wsems[s])
            pltpu.async_copy(ebuf.at[s, 1], pv_hbm.at[qa], wsems[s])

        def wait_writes(s):
            pltpu.make_async_copy(ebuf.at[s, 0], pk_hbm.at[0],
                                  wsems[s]).wait()
            pltpu.make_async_copy(ebuf.at[s, 1], pv_hbm.at[0],
                                  wsems[s]).wait()

        for s in range(nslots):
            issue_gather(s, s)

        @pl.loop(0, qpw, step=nslots)
        def _(q0):
            for s in range(nslots):
                wait_gather(s)
                issue_writes(q0 + s, s)
            for s in range(nslots):
                wait_writes(s)
                nxt = q0 + nslots + s

                @pl.when(nxt < qpw)
                def _():
                    issue_gather(nxt, s)

    return k


def kernel(x_querry, l, x_block, e_k, e_p_0, g_p_0, train_count):
    B, _ = x_querry.shape
    E, PLEN, D = e_p_0.shape
    idx = _routing_call(x_querry, e_k, train_count.reshape(1, E))
    gpad = jnp.zeros((2, 12, D), jnp.float32)
    gpad = gpad.at[0, 8:12].set(g_p_0[:4]).at[1, 8:12].set(g_p_0[4:])
    pk, pv = _gather_kernel(B, D)(
        e_p_0.reshape(E * PLEN, D), gpad, idx.reshape(_NW, 1, 16 * B // _NW))
    return pk, pv, x_block
